# Initial kernel scaffold; baseline (speedup 1.0000x reference)
import jax
import jax.numpy as jnp
from jax.experimental import pallas as pl


def kernel(x, cluster_centers):
    raise NotImplementedError("write your pallas kernel here")


# speedup vs baseline: 2.6616x; 2.6616x over previous
import jax
import jax.numpy as jnp
from jax.experimental import pallas as pl
from jax.experimental.pallas import tpu as pltpu

N_PIX = 262144
K = 1024
BLOCK = 1024


def _vq_body(x_ref, ct_ref, ckc_ref, o_ref):
    x0 = x_ref[:, 0:1]
    x1 = x_ref[:, 1:2]
    x2 = x_ref[:, 2:3]
    d0 = x0 - ct_ref[0:1, :]
    d1 = x1 - ct_ref[1:2, :]
    d2 = x2 - ct_ref[2:3, :]
    d = d0 * d0 + d1 * d1 + d2 * d2
    idx = jnp.argmin(d, axis=1)
    onehot = (jax.lax.broadcasted_iota(jnp.int32, (BLOCK, K), 1)
              == idx[:, None]).astype(jnp.float32)
    o_ref[...] = jnp.dot(onehot, ckc_ref[...],
                         preferred_element_type=jnp.float32)


def kernel(x, cluster_centers):
    ct = cluster_centers.T
    grid = (N_PIX // BLOCK,)
    return pl.pallas_call(
        _vq_body,
        grid=grid,
        in_specs=[
            pl.BlockSpec((BLOCK, 3), lambda i: (i, 0)),
            pl.BlockSpec((3, K), lambda i: (0, 0)),
            pl.BlockSpec((K, 3), lambda i: (0, 0)),
        ],
        out_specs=pl.BlockSpec((BLOCK, 3), lambda i: (i, 0)),
        out_shape=jax.ShapeDtypeStruct((N_PIX, 3), jnp.float32),
        compiler_params=pltpu.CompilerParams(
            dimension_semantics=("arbitrary",),
        ),
    )(x, ct, cluster_centers)
